# baseline (speedup 1.0000x reference)
import jax
import jax.numpy as jnp
from jax.experimental import pallas as pl
from jax.experimental.pallas import tpu as pltpu

_ALPHA = 0.5
_LOG2E = 1.4426950408889634


def _gat_kernel(adj_ref, h_ref, w_ref, a1_ref, a2_ref, out_ref,
                wh_ref, whaug_ref, w2c_ref, w2ca_ref, csum_ref, m2_ref):
    i = pl.program_id(0)
    r = out_ref.shape[0]
    n = adj_ref.shape[1]
    c = out_ref.shape[1]

    @pl.when(i == 0)
    def _():
        wh = jnp.dot(h_ref[...], w_ref[...],
                     preferred_element_type=jnp.float32)
        wh_ref[...] = wh
        whaug_ref[:, 0:c] = wh.astype(jnp.bfloat16)
        whaug_ref[:, c:c + 1] = jnp.ones((n, 1), dtype=jnp.bfloat16)
        whaug_ref[:, c + 1:2 * c] = jnp.zeros((n, c - 1), dtype=jnp.bfloat16)
        w2 = jax.lax.dot_general(a2_ref[...], wh,
                                 (((1,), (1,)), ((), ())),
                                 preferred_element_type=jnp.float32)
        m2_ref[...] = jnp.max(w2, axis=(0, 1), keepdims=True)
        w2c_ref[...] = w2 * _LOG2E
        w2ca_ref[...] = w2 * (_ALPHA * _LOG2E)
        ones = jnp.ones((1, n), dtype=jnp.float32)
        csum_ref[...] = jax.lax.dot_general(ones, wh,
                                            (((1,), (0,)), ((), ())),
                                            preferred_element_type=jnp.float32)

    wh_blk = wh_ref[pl.ds(i * r, r), :]
    wh1 = jax.lax.dot_general(wh_blk, a1_ref[...],
                              (((1,), (1,)), ((), ())),
                              preferred_element_type=jnp.float32)
    u = wh1 + m2_ref[...]
    mi = jnp.maximum(u, _ALPHA * u)
    q1 = _LOG2E * (wh1 - mi)
    q2 = _LOG2E * (_ALPHA * wh1 - mi)
    arg = jnp.maximum(q1 + w2c_ref[...], q2 + w2ca_ref[...])
    p = adj_ref[...] * jnp.exp2(arg)
    pb = p.astype(jnp.bfloat16)
    pm = jax.lax.dot_general(pb, whaug_ref[...], (((1,), (0,)), ((), ())),
                             preferred_element_type=jnp.float32)
    s = pm[:, c:c + 1]
    safe = jnp.where(s > 0, s, 1.0)
    hp = jnp.where(s > 0, pm[:, :c] / safe, csum_ref[...] / n)
    out_ref[...] = jnp.maximum(hp, 0.0)


def kernel(h, adj, W, a):
    n, in_ch = h.shape
    out_ch = W.shape[1]
    a1 = a[:out_ch].reshape(1, out_ch)
    a2 = a[out_ch:].reshape(1, out_ch)

    r = 200 if n % 200 == 0 else n
    out = pl.pallas_call(
        _gat_kernel,
        grid=(n // r,),
        in_specs=[pl.BlockSpec((r, n), lambda i: (i, 0)),
                  pl.BlockSpec((n, in_ch), lambda i: (0, 0)),
                  pl.BlockSpec((in_ch, out_ch), lambda i: (0, 0)),
                  pl.BlockSpec((1, out_ch), lambda i: (0, 0)),
                  pl.BlockSpec((1, out_ch), lambda i: (0, 0))],
        out_specs=pl.BlockSpec((r, out_ch), lambda i: (i, 0)),
        out_shape=jax.ShapeDtypeStruct((n, out_ch), jnp.float32),
        scratch_shapes=[pltpu.VMEM((n, out_ch), jnp.float32),
                        pltpu.VMEM((n, 2 * out_ch), jnp.bfloat16),
                        pltpu.VMEM((1, n), jnp.float32),
                        pltpu.VMEM((1, n), jnp.float32),
                        pltpu.VMEM((1, out_ch), jnp.float32),
                        pltpu.VMEM((1, 1), jnp.float32)],
    )(adj, h, W, a1, a2)
    return out


# speedup vs baseline: 1.1216x; 1.1216x over previous
import jax
import jax.numpy as jnp
from jax.experimental import pallas as pl
from jax.experimental.pallas import tpu as pltpu

_ALPHA = 0.5
_LOG2E = 1.4426950408889634


def _gat_kernel(adj_ref, h_ref, w_ref, a1_ref, a2_ref, out_ref,
                wh_ref, whaug_ref, w2c_ref, w2ca_ref, csum_ref, m2_ref):
    i = pl.program_id(0)
    r = out_ref.shape[0]
    n = adj_ref.shape[1]
    c = out_ref.shape[1]

    @pl.when(i == 0)
    def _():
        wh = jnp.dot(h_ref[...], w_ref[...],
                     preferred_element_type=jnp.float32)
        wh_ref[...] = wh
        whaug_ref[:, 0:c] = wh.astype(jnp.bfloat16)
        whaug_ref[:, c:c + 1] = jnp.ones((n, 1), dtype=jnp.bfloat16)
        whaug_ref[:, c + 1:2 * c] = jnp.zeros((n, c - 1), dtype=jnp.bfloat16)
        w2 = jax.lax.dot_general(a2_ref[...], wh,
                                 (((1,), (1,)), ((), ())),
                                 preferred_element_type=jnp.float32)
        m2_ref[...] = jnp.max(w2, axis=(0, 1), keepdims=True)
        w2c_ref[...] = w2 * _LOG2E
        w2ca_ref[...] = w2 * (_ALPHA * _LOG2E)
        ones = jnp.ones((1, n), dtype=jnp.float32)
        csum_ref[...] = jax.lax.dot_general(ones, wh,
                                            (((1,), (0,)), ((), ())),
                                            preferred_element_type=jnp.float32)

    wh_blk = wh_ref[pl.ds(i * r, r), :]
    wh1 = jax.lax.dot_general(wh_blk, a1_ref[...],
                              (((1,), (1,)), ((), ())),
                              preferred_element_type=jnp.float32)
    u = wh1 + m2_ref[...]
    mi = jnp.maximum(u, _ALPHA * u)
    q1 = _LOG2E * (wh1 - mi)
    q2 = _LOG2E * (_ALPHA * wh1 - mi)
    arg = jnp.maximum(q1 + w2c_ref[...], q2 + w2ca_ref[...])
    p = adj_ref[...] * jnp.exp2(arg)
    pb = p.astype(jnp.bfloat16)
    pm = jax.lax.dot_general(pb, whaug_ref[...], (((1,), (0,)), ((), ())),
                             preferred_element_type=jnp.float32)
    s = pm[:, c:c + 1]
    safe = jnp.where(s > 0, s, 1.0)
    hp = jnp.where(s > 0, pm[:, :c] / safe, csum_ref[...] / n)
    out_ref[...] = jnp.maximum(hp, 0.0)


def kernel(h, adj, W, a):
    n, in_ch = h.shape
    out_ch = W.shape[1]
    a1 = a[:out_ch].reshape(1, out_ch)
    a2 = a[out_ch:].reshape(1, out_ch)

    r = 400 if n % 400 == 0 else n
    out = pl.pallas_call(
        _gat_kernel,
        grid=(n // r,),
        in_specs=[pl.BlockSpec((r, n), lambda i: (i, 0)),
                  pl.BlockSpec((n, in_ch), lambda i: (0, 0)),
                  pl.BlockSpec((in_ch, out_ch), lambda i: (0, 0)),
                  pl.BlockSpec((1, out_ch), lambda i: (0, 0)),
                  pl.BlockSpec((1, out_ch), lambda i: (0, 0))],
        out_specs=pl.BlockSpec((r, out_ch), lambda i: (i, 0)),
        out_shape=jax.ShapeDtypeStruct((n, out_ch), jnp.float32),
        scratch_shapes=[pltpu.VMEM((n, out_ch), jnp.float32),
                        pltpu.VMEM((n, 2 * out_ch), jnp.bfloat16),
                        pltpu.VMEM((1, n), jnp.float32),
                        pltpu.VMEM((1, n), jnp.float32),
                        pltpu.VMEM((1, out_ch), jnp.float32),
                        pltpu.VMEM((1, 1), jnp.float32)],
    )(adj, h, W, a1, a2)
    return out
